# Initial kernel scaffold; baseline (speedup 1.0000x reference)
import jax
import jax.numpy as jnp
from jax.experimental import pallas as pl


def kernel(exp, x, edge_index, edge_attr, batch, params):
    raise NotImplementedError("write your pallas kernel here")


# speedup vs baseline: 11.3786x; 11.3786x over previous
import jax
import jax.numpy as jnp
from jax import lax
from jax.experimental import pallas as pl
from jax.experimental.pallas import tpu as pltpu
from jax.experimental.pallas import tpu_sc as plsc

N = 10000
E = 320000
B = 128
DF = 128
DE = 16
DM = 1024
DG = 2089
DGE = 512
HID = 512
OUT = 256

W = 32
EPW = E // W
C = 80
NCH = EPW // C
QR = 1280
EB = 32768

R = 1000
G = N // R


def _prep_body(x_ref, wn_ref, a2_ref, hs_ref, hd_ref):
    w2 = jnp.dot(wn_ref[...], a2_ref[...], preferred_element_type=jnp.float32)
    h2 = lax.dot_general(w2, x_ref[...], (((0,), (1,)), ((), ())),
                         preferred_element_type=jnp.float32)
    hs_ref[...] = h2[0]
    hd_ref[...] = h2[1]


def _el_body(ea_ref, wet_ref, aer_ref, el_ref):
    wea = jnp.dot(aer_ref[...], wet_ref[...], preferred_element_type=jnp.float32)
    el_ref[...] = jnp.sum(ea_ref[...] * wea, axis=1)


def _logit_sc(esrc_hbm, edst_hbm, hs_hbm, hd_hbm, el_hbm,
              e_out, s_out,
              hs_v, hd_v, src_v, dst_v, el_v, e_v, s_acc):
    cid = lax.axis_index("c")
    sid = lax.axis_index("s")
    wid = sid * 2 + cid
    ebase = wid * EPW

    pltpu.sync_copy(hs_hbm, hs_v)
    pltpu.sync_copy(hd_hbm, hd_v)

    z16 = jnp.zeros((16,), jnp.float32)

    def zs(i, c):
        s_acc[pl.ds(i * 16, 16)] = z16
        return c
    lax.fori_loop(0, N // 16, zs, 0)

    def chunk(k, c):
        off = ebase + k * C
        pltpu.sync_copy(esrc_hbm.at[pl.ds(off, C)], src_v)
        pltpu.sync_copy(edst_hbm.at[pl.ds(off, C)], dst_v)
        pltpu.sync_copy(el_hbm.at[pl.ds(off, C)], el_v)
        for j in range(C // 16):
            sl = pl.ds(j * 16, 16)
            ids = src_v[sl]
            idd = dst_v[sl]
            gs = plsc.load_gather(hs_v, [ids])
            gd = plsc.load_gather(hd_v, [idd])
            logit = gs + gd + el_v[sl]
            logit = jnp.where(logit >= 0.0, logit, 0.2 * logit)
            e = jnp.exp(logit)
            e_v[sl] = e
            plsc.addupdate_scatter(s_acc, [idd], e)
        pltpu.sync_copy(e_v, e_out.at[pl.ds(off, C)])
        return c
    lax.fori_loop(0, NCH, chunk, 0)

    pltpu.sync_copy(s_acc, s_out.at[pl.ds(wid * N, N)])


def _scatter_sc(esrc_hbm, edst_hbm, e_hbm, x_hbm, ea_hbm,
                p0_out, p1_out, q0_out, q1_out,
                src_v, dst_v, qdst_v, e_v, ea_v, rows_v, qrow_v,
                p_sp, q_sp, sem_r):
    cid = lax.axis_index("c")
    sid = lax.axis_index("s")
    wid = sid * 2 + cid
    ebase = wid * EPW

    z16 = jnp.zeros((16,), jnp.float32)

    def zr(i, c):
        for f in range(8):
            rows_v[i, pl.ds(f * 16, 16)] = z16
            qrow_v[i, pl.ds(f * 16, 16)] = z16
        ea_v[i, :] = z16
        return c
    lax.fori_loop(0, C, zr, 0)

    def zcp(t, c):
        @pl.when(t % 16 == sid)
        def _():
            pltpu.sync_copy(rows_v, p_sp.at[pl.ds(t * C, C)])
        return c
    lax.fori_loop(0, N // C, zcp, 0)

    @pl.when(sid < QR // C)
    def _():
        pltpu.sync_copy(qrow_v, q_sp.at[pl.ds(sid * C, C)])

    plsc.subcore_barrier()

    def chunk(k, c):
        off = ebase + k * C
        pltpu.sync_copy(esrc_hbm.at[pl.ds(off, C)], src_v)
        pltpu.sync_copy(edst_hbm.at[pl.ds(off, C)], dst_v)
        pltpu.sync_copy(e_hbm.at[pl.ds(off, C)], e_v)
        pltpu.sync_copy(ea_hbm.at[pl.ds(off, C)], ea_v)
        cp_rows = pltpu.async_copy(x_hbm.at[src_v], rows_v, sem_r)
        for j in range(C // 16):
            sl = pl.ds(j * 16, 16)
            qdst_v[sl] = lax.shift_right_logical(dst_v[sl], 3)
        cp_rows.wait()

        def scale(j, c2):
            ev16 = e_v[pl.ds(j * 16, 16)]
            dv16 = dst_v[pl.ds(j * 16, 16)]
            for k2 in range(16):
                i = j * 16 + k2
                ev = jnp.full((16,), ev16[k2], jnp.float32)
                base = (dv16[k2] & 7) * 16
                qrow_v[i, pl.ds(base, 16)] = ea_v[i, :] * ev
                for f in range(8):
                    rows_v[i, pl.ds(f * 16, 16)] = (
                        rows_v[i, pl.ds(f * 16, 16)] * ev)
            return c2
        lax.fori_loop(0, C // 16, scale, 0)

        pltpu.sync_copy(rows_v, p_sp.at[dst_v], add=True)
        pltpu.sync_copy(qrow_v, q_sp.at[qdst_v], add=True)

        def unscale(j, c2):
            dv16 = dst_v[pl.ds(j * 16, 16)]
            for k2 in range(16):
                base = (dv16[k2] & 7) * 16
                qrow_v[j * 16 + k2, pl.ds(base, 16)] = z16
            return c2
        lax.fori_loop(0, C // 16, unscale, 0)
        return c
    lax.fori_loop(0, NCH, chunk, 0)

    plsc.subcore_barrier()

    def drain(t, c):
        @pl.when(t % 16 == sid)
        def _():
            r0 = t * C
            pltpu.sync_copy(p_sp.at[pl.ds(r0, C)], rows_v)

            @pl.when(cid == 0)
            def _():
                pltpu.sync_copy(rows_v, p0_out.at[pl.ds(r0, C)])

            @pl.when(cid == 1)
            def _():
                pltpu.sync_copy(rows_v, p1_out.at[pl.ds(r0, C)])
        return c
    lax.fori_loop(0, N // C, drain, 0)

    @pl.when(sid < QR // C)
    def _():
        pltpu.sync_copy(q_sp.at[pl.ds(sid * C, C)], qrow_v)

        @pl.when(cid == 0)
        def _():
            pltpu.sync_copy(qrow_v, q0_out.at[pl.ds(sid * C, C)])

        @pl.when(cid == 1)
        def _():
            pltpu.sync_copy(qrow_v, q1_out.at[pl.ds(sid * C, C)])


def _gene_body(exp_ref, wg_ref, bg_ref, wwg_ref, bwg_ref, expe_ref, xg_ref):
    ee = jnp.dot(exp_ref[...], wg_ref[...],
                 preferred_element_type=jnp.float32) + bg_ref[...]
    expe_ref[...] = ee
    xg_ref[...] = jnp.dot(jnp.maximum(ee, 0.0), wwg_ref[...],
                          preferred_element_type=jnp.float32) + bwg_ref[...]


def _main_body(p0_ref, p1_ref, pe0_ref, pe1_ref, st_ref,
               batch_ref, xg_ref, wn_ref, we_ref,
               wgs_ref, bgs_ref, wcat_ref, bcat_ref, wg2_ref, bg2_ref,
               s1_out, c1_out, s2_out, c2_out,
               s1_acc, c1_acc, s2_acc, c2_acc):
    i = pl.program_id(0)

    @pl.when(i == 0)
    def _():
        s1_acc[...] = jnp.zeros_like(s1_acc)
        c1_acc[...] = jnp.zeros_like(c1_acc)
        s2_acc[...] = jnp.zeros_like(s2_acc)
        c2_acc[...] = jnp.zeros_like(c2_acc)

    pmsg = p0_ref[...] + p1_ref[...]
    pemsg = pe0_ref[...] + pe1_ref[...]
    s = jnp.sum(st_ref[...], axis=1)
    inv = 1.0 / (s + 1e-16)
    ne = jnp.dot(pmsg, wn_ref[...], preferred_element_type=jnp.float32)
    ne = ne + jnp.dot(pemsg, we_ref[...], preferred_element_type=jnp.float32)
    ne = ne * inv[:, None]
    ne = jnp.where(ne > 0.0, ne, jnp.exp(jnp.minimum(ne, 0.0)) - 1.0)

    b = batch_ref[0, 0, :]
    onehot = (b[:, None] == lax.broadcasted_iota(jnp.int32, (R, B), 1)
              ).astype(jnp.float32)
    z = ne + jnp.dot(onehot, xg_ref[...], preferred_element_type=jnp.float32)

    gs = jnp.sum(ne * wgs_ref[...], axis=1) + bgs_ref[0, 0]
    es = jnp.exp(gs)

    uv = jnp.dot(z, wcat_ref[...], preferred_element_type=jnp.float32)
    uv = jnp.maximum(uv + bcat_ref[...], 0.0)
    u1 = uv[:, :HID]
    u2 = uv[:, HID:]
    g = jnp.sum(u1 * wg2_ref[...], axis=1) + bg2_ref[0, 0]
    eg = jnp.exp(g)

    cdims = (((0,), (0,)), ((), ()))
    s1_acc[...] += lax.dot_general(onehot, es[:, None] * ne, cdims,
                                   preferred_element_type=jnp.float32)
    c1_acc[...] += lax.dot_general(onehot, es[:, None], cdims,
                                   preferred_element_type=jnp.float32)
    s2_acc[...] += lax.dot_general(onehot, eg[:, None] * u2, cdims,
                                   preferred_element_type=jnp.float32)
    c2_acc[...] += lax.dot_general(onehot, eg[:, None], cdims,
                                   preferred_element_type=jnp.float32)

    @pl.when(i == G - 1)
    def _():
        s1_out[...] = s1_acc[...]
        c1_out[...] = c1_acc[...]
        s2_out[...] = s2_acc[...]
        c2_out[...] = c2_acc[...]


def _heads_body(s1, c1, s2, c2, expe,
                wns, bns, wn2, bn2,
                wd1, bd1, wd2, bd2,
                wl1, bl1, wl2, bl2,
                wi1, bi1, wi2, bi2, out_ref):
    def fc(v, w1, b1, w2, b2):
        hh = jnp.dot(jnp.maximum(v, 0.0), w1[...],
                     preferred_element_type=jnp.float32) + b1[...]
        sg = 1.0 / (1.0 + jnp.exp(-hh))
        return jnp.dot(sg, w2[...], preferred_element_type=jnp.float32) + b2[...]

    r1 = 1.0 / (c1[...] + 1e-16)
    xd = jnp.dot(s1[...], wns[...],
                 preferred_element_type=jnp.float32) * r1 + (c1[...] * r1) * bns[...]
    r2 = 1.0 / (c2[...] + 1e-16)
    xi = jnp.dot(s2[...], wn2[...],
                 preferred_element_type=jnp.float32) * r2 + (c2[...] * r2) * bn2[...]
    o1 = fc(xd, wd1, bd1, wd2, bd2)
    o2 = fc(expe[...], wl1, bl1, wl2, bl2)
    o3 = fc(xi, wi1, bi1, wi2, bi2)
    out_ref[...] = jnp.concatenate([o1, o2, o3], axis=1)


def _f32(shape):
    return jax.ShapeDtypeStruct(shape, jnp.float32)


def kernel(exp, x, edge_index, edge_attr, batch, params):
    p = params

    a2 = jnp.stack([p['a_src'], p['a_dst']], axis=1)
    hs, hd = pl.pallas_call(
        _prep_body,
        out_shape=[_f32((N,)), _f32((N,))],
    )(x, p['Wn'], a2)

    el = pl.pallas_call(
        _el_body,
        grid=(pl.cdiv(E, EB),),
        in_specs=[
            pl.BlockSpec((EB, DE), lambda i: (i, 0)),
            pl.BlockSpec((DM, DE), lambda i: (0, 0)),
            pl.BlockSpec((1, DM), lambda i: (0, 0)),
        ],
        out_specs=pl.BlockSpec((EB,), lambda i: (i,)),
        out_shape=_f32((E,)),
    )(edge_attr, p['We'].T, p['a_e'].reshape(1, DM))

    sc_mesh = plsc.VectorSubcoreMesh(core_axis_name="c", subcore_axis_name="s",
                                     num_cores=2, num_subcores=16)
    logit_fn = pl.kernel(
        _logit_sc,
        out_type=[_f32((E,)), _f32((W * N,))],
        mesh=sc_mesh,
        compiler_params=pltpu.CompilerParams(needs_layout_passes=False),
        scratch_types=[
            pltpu.VMEM((N,), jnp.float32),
            pltpu.VMEM((N,), jnp.float32),
            pltpu.VMEM((C,), jnp.int32),
            pltpu.VMEM((C,), jnp.int32),
            pltpu.VMEM((C,), jnp.float32),
            pltpu.VMEM((C,), jnp.float32),
            pltpu.VMEM((N,), jnp.float32),
        ],
    )
    ew, s_flat = logit_fn(edge_index[0], edge_index[1], hs, hd, el)

    scatter_fn = pl.kernel(
        _scatter_sc,
        out_type=[_f32((N, DF)), _f32((N, DF)),
                  _f32((QR, DF)), _f32((QR, DF))],
        mesh=sc_mesh,
        compiler_params=pltpu.CompilerParams(needs_layout_passes=False),
        scratch_types=[
            pltpu.VMEM((C,), jnp.int32),
            pltpu.VMEM((C,), jnp.int32),
            pltpu.VMEM((C,), jnp.int32),
            pltpu.VMEM((C,), jnp.float32),
            pltpu.VMEM((C, DE), jnp.float32),
            pltpu.VMEM((C, DF), jnp.float32),
            pltpu.VMEM((C, DF), jnp.float32),
            pltpu.VMEM_SHARED((N, DF), jnp.float32),
            pltpu.VMEM_SHARED((QR, DF), jnp.float32),
            pltpu.SemaphoreType.DMA,
        ],
    )
    p0, p1, q0, q1 = scatter_fn(edge_index[0], edge_index[1], ew, x, edge_attr)

    pe0 = q0[:N // 8].reshape(N, DE)
    pe1 = q1[:N // 8].reshape(N, DE)
    s_t = jnp.transpose(s_flat.reshape(W, N))

    expe, xg = pl.pallas_call(
        _gene_body,
        out_shape=[_f32((B, DGE)), _f32((B, DM))],
    )(exp, p['Wg'], p['bg'].reshape(1, -1), p['Wwg'], p['bwg'].reshape(1, -1))

    wcat = jnp.concatenate([p['Wg1'], p['Wn1']], axis=1)
    bcat = jnp.concatenate([p['bg1'], p['bn1']]).reshape(1, -1)
    batch3 = batch.reshape(G, 1, R)

    s1, c1, s2, c2 = pl.pallas_call(
        _main_body,
        grid=(G,),
        in_specs=[
            pl.BlockSpec((R, DF), lambda i: (i, 0)),
            pl.BlockSpec((R, DF), lambda i: (i, 0)),
            pl.BlockSpec((R, DE), lambda i: (i, 0)),
            pl.BlockSpec((R, DE), lambda i: (i, 0)),
            pl.BlockSpec((R, W), lambda i: (i, 0)),
            pl.BlockSpec((1, 1, R), lambda i: (i, 0, 0)),
            pl.BlockSpec((B, DM), lambda i: (0, 0)),
            pl.BlockSpec((DF, DM), lambda i: (0, 0)),
            pl.BlockSpec((DE, DM), lambda i: (0, 0)),
            pl.BlockSpec((1, DM), lambda i: (0, 0)),
            pl.BlockSpec((1, 1), lambda i: (0, 0)),
            pl.BlockSpec((DM, 2 * HID), lambda i: (0, 0)),
            pl.BlockSpec((1, 2 * HID), lambda i: (0, 0)),
            pl.BlockSpec((1, HID), lambda i: (0, 0)),
            pl.BlockSpec((1, 1), lambda i: (0, 0)),
        ],
        out_specs=[
            pl.BlockSpec((B, DM), lambda i: (0, 0)),
            pl.BlockSpec((B, 1), lambda i: (0, 0)),
            pl.BlockSpec((B, HID), lambda i: (0, 0)),
            pl.BlockSpec((B, 1), lambda i: (0, 0)),
        ],
        out_shape=[_f32((B, DM)), _f32((B, 1)), _f32((B, HID)), _f32((B, 1))],
        scratch_shapes=[
            pltpu.VMEM((B, DM), jnp.float32),
            pltpu.VMEM((B, 1), jnp.float32),
            pltpu.VMEM((B, HID), jnp.float32),
            pltpu.VMEM((B, 1), jnp.float32),
        ],
    )(p0, p1, pe0, pe1, s_t,
      batch3, xg, p['Wn'], p['We'],
      p['Wgs'].reshape(1, -1), p['bgs'].reshape(1, 1),
      wcat, bcat, p['Wg2'].reshape(1, -1), p['bg2'].reshape(1, 1))

    out = pl.pallas_call(
        _heads_body,
        out_shape=_f32((B, 3)),
    )(s1, c1, s2, c2, expe,
      p['Wns'], p['bns'].reshape(1, -1), p['Wn2'], p['bn2'].reshape(1, -1),
      p['Wd1'], p['bd1'].reshape(1, -1), p['Wd2'], p['bd2'].reshape(1, 1),
      p['Wl1'], p['bl1'].reshape(1, -1), p['Wl2'], p['bl2'].reshape(1, 1),
      p['Wi1'], p['bi1'].reshape(1, -1), p['Wi2'], p['bi2'].reshape(1, 1))
    return out


# speedup vs baseline: 15.6025x; 1.3712x over previous
import jax
import jax.numpy as jnp
from jax import lax
from jax.experimental import pallas as pl
from jax.experimental.pallas import tpu as pltpu
from jax.experimental.pallas import tpu_sc as plsc

N = 10000
E = 320000
B = 128
DF = 128
DE = 16
DM = 1024
DG = 2089
DGE = 512
HID = 512
OUT = 256

W = 32
EPW = E // W
C = 80
NCH = EPW // C
QR = 1280
EB = 32768

R = 1000
G = N // R


def _prep_body(x_ref, wn_ref, a2_ref, hs_ref, hd_ref):
    w2 = jnp.dot(wn_ref[...], a2_ref[...], preferred_element_type=jnp.float32)
    h2 = lax.dot_general(w2, x_ref[...], (((0,), (1,)), ((), ())),
                         preferred_element_type=jnp.float32)
    hs_ref[...] = h2[0]
    hd_ref[...] = h2[1]


def _el_body(ea_ref, wet_ref, aer_ref, el_ref):
    wea = jnp.dot(aer_ref[...], wet_ref[...], preferred_element_type=jnp.float32)
    el_ref[...] = jnp.sum(ea_ref[...] * wea, axis=1)


def _logit_sc(esrc_hbm, edst_hbm, hs_hbm, hd_hbm, el_hbm,
              e_out, s_out,
              hs_v, hd_v, src_v, dst_v, el_v, e_v, s_acc,
              sem_a, sem_b, sem_c):
    cid = lax.axis_index("c")
    sid = lax.axis_index("s")
    wid = sid * 2 + cid
    ebase = wid * EPW

    pltpu.sync_copy(hs_hbm, hs_v)
    pltpu.sync_copy(hd_hbm, hd_v)

    z16 = jnp.zeros((16,), jnp.float32)

    def zs(i, c):
        s_acc[pl.ds(i * 16, 16)] = z16
        return c
    lax.fori_loop(0, N // 16, zs, 0)

    def chunk(k, c):
        off = ebase + k * C
        cp_a = pltpu.async_copy(esrc_hbm.at[pl.ds(off, C)], src_v, sem_a)
        cp_b = pltpu.async_copy(edst_hbm.at[pl.ds(off, C)], dst_v, sem_b)
        cp_c = pltpu.async_copy(el_hbm.at[pl.ds(off, C)], el_v, sem_c)
        cp_a.wait()
        cp_b.wait()
        cp_c.wait()
        for j in range(C // 16):
            sl = pl.ds(j * 16, 16)
            ids = src_v[sl]
            idd = dst_v[sl]
            gs = plsc.load_gather(hs_v, [ids])
            gd = plsc.load_gather(hd_v, [idd])
            logit = gs + gd + el_v[sl]
            logit = jnp.where(logit >= 0.0, logit, 0.2 * logit)
            e = jnp.exp(logit)
            e_v[sl] = e
            plsc.addupdate_scatter(s_acc, [idd], e)
        pltpu.sync_copy(e_v, e_out.at[pl.ds(off, C)])
        return c
    lax.fori_loop(0, NCH, chunk, 0)

    pltpu.sync_copy(s_acc, s_out.at[pl.ds(wid * N, N)])


def _scatter_sc(esrc_hbm, edst_hbm, e_hbm, x_hbm, ea_hbm,
                p0_out, p1_out, q0_out, q1_out,
                src_v, dst_v, qdst_v, e_v, ea_v, rows_v, qrow_v,
                p_sp, q_sp, sem_r, sem_a, sem_b, sem_c, sem_d):
    cid = lax.axis_index("c")
    sid = lax.axis_index("s")
    wid = sid * 2 + cid
    ebase = wid * EPW

    z16 = jnp.zeros((16,), jnp.float32)

    def zr(i, c):
        for f in range(8):
            rows_v[i, pl.ds(f * 16, 16)] = z16
            qrow_v[i, pl.ds(f * 16, 16)] = z16
        ea_v[i, :] = z16
        return c
    lax.fori_loop(0, C, zr, 0)

    def zcp(t, c):
        @pl.when(t % 16 == sid)
        def _():
            pltpu.sync_copy(rows_v, p_sp.at[pl.ds(t * C, C)])
        return c
    lax.fori_loop(0, N // C, zcp, 0)

    @pl.when(sid < QR // C)
    def _():
        pltpu.sync_copy(qrow_v, q_sp.at[pl.ds(sid * C, C)])

    plsc.subcore_barrier()

    def chunk(k, c):
        off = ebase + k * C
        cp_a = pltpu.async_copy(esrc_hbm.at[pl.ds(off, C)], src_v, sem_a)
        cp_b = pltpu.async_copy(edst_hbm.at[pl.ds(off, C)], dst_v, sem_b)
        cp_c = pltpu.async_copy(e_hbm.at[pl.ds(off, C)], e_v, sem_c)
        cp_d = pltpu.async_copy(ea_hbm.at[pl.ds(off, C)], ea_v, sem_d)
        cp_a.wait()
        cp_rows = pltpu.async_copy(x_hbm.at[src_v], rows_v, sem_r)
        cp_b.wait()
        cp_c.wait()
        cp_d.wait()
        for j in range(C // 16):
            sl = pl.ds(j * 16, 16)
            qdst_v[sl] = lax.shift_right_logical(dst_v[sl], 3)
        cp_rows.wait()

        def scale(j, c2):
            ev16 = e_v[pl.ds(j * 16, 16)]
            dv16 = dst_v[pl.ds(j * 16, 16)]
            for k2 in range(16):
                i = j * 16 + k2
                ev = jnp.full((16,), ev16[k2], jnp.float32)
                base = (dv16[k2] & 7) * 16
                qrow_v[i, pl.ds(base, 16)] = ea_v[i, :] * ev
                for f in range(8):
                    rows_v[i, pl.ds(f * 16, 16)] = (
                        rows_v[i, pl.ds(f * 16, 16)] * ev)
            return c2
        lax.fori_loop(0, C // 16, scale, 0)

        pltpu.sync_copy(rows_v, p_sp.at[dst_v], add=True)
        pltpu.sync_copy(qrow_v, q_sp.at[qdst_v], add=True)

        def unscale(j, c2):
            dv16 = dst_v[pl.ds(j * 16, 16)]
            for k2 in range(16):
                base = (dv16[k2] & 7) * 16
                qrow_v[j * 16 + k2, pl.ds(base, 16)] = z16
            return c2
        lax.fori_loop(0, C // 16, unscale, 0)
        return c
    lax.fori_loop(0, NCH, chunk, 0)

    plsc.subcore_barrier()

    def drain(t, c):
        @pl.when(t % 16 == sid)
        def _():
            r0 = t * C
            pltpu.sync_copy(p_sp.at[pl.ds(r0, C)], rows_v)

            @pl.when(cid == 0)
            def _():
                pltpu.sync_copy(rows_v, p0_out.at[pl.ds(r0, C)])

            @pl.when(cid == 1)
            def _():
                pltpu.sync_copy(rows_v, p1_out.at[pl.ds(r0, C)])
        return c
    lax.fori_loop(0, N // C, drain, 0)

    @pl.when(sid < QR // C)
    def _():
        pltpu.sync_copy(q_sp.at[pl.ds(sid * C, C)], qrow_v)

        @pl.when(cid == 0)
        def _():
            pltpu.sync_copy(qrow_v, q0_out.at[pl.ds(sid * C, C)])

        @pl.when(cid == 1)
        def _():
            pltpu.sync_copy(qrow_v, q1_out.at[pl.ds(sid * C, C)])


def _gene_body(exp_ref, wg_ref, bg_ref, wwg_ref, bwg_ref, expe_ref, xg_ref):
    ee = jnp.dot(exp_ref[...], wg_ref[...],
                 preferred_element_type=jnp.float32) + bg_ref[...]
    expe_ref[...] = ee
    xg_ref[...] = jnp.dot(jnp.maximum(ee, 0.0), wwg_ref[...],
                          preferred_element_type=jnp.float32) + bwg_ref[...]


def _main_body(p0_ref, p1_ref, pe0_ref, pe1_ref, st_ref,
               batch_ref, xg_ref, wn_ref, we_ref,
               wgs_ref, bgs_ref, wcat_ref, bcat_ref, wg2_ref, bg2_ref,
               s1_out, c1_out, s2_out, c2_out,
               s1_acc, c1_acc, s2_acc, c2_acc):
    i = pl.program_id(0)

    @pl.when(i == 0)
    def _():
        s1_acc[...] = jnp.zeros_like(s1_acc)
        c1_acc[...] = jnp.zeros_like(c1_acc)
        s2_acc[...] = jnp.zeros_like(s2_acc)
        c2_acc[...] = jnp.zeros_like(c2_acc)

    pmsg = p0_ref[...] + p1_ref[...]
    pemsg = pe0_ref[...] + pe1_ref[...]
    s = jnp.sum(st_ref[...], axis=1)
    inv = 1.0 / (s + 1e-16)
    bf = jnp.bfloat16
    ne = jnp.dot(pmsg.astype(bf), wn_ref[...].astype(bf),
                 preferred_element_type=jnp.float32)
    ne = ne + jnp.dot(pemsg.astype(bf), we_ref[...].astype(bf),
                      preferred_element_type=jnp.float32)
    ne = ne * inv[:, None]
    ne = jnp.where(ne > 0.0, ne, jnp.exp(jnp.minimum(ne, 0.0)) - 1.0)

    b = batch_ref[0, 0, :]
    onehot = (b[:, None] == lax.broadcasted_iota(jnp.int32, (R, B), 1)
              ).astype(jnp.float32)
    z = ne + jnp.dot(onehot.astype(bf), xg_ref[...].astype(bf),
                     preferred_element_type=jnp.float32)

    gs = jnp.sum(ne * wgs_ref[...], axis=1) + bgs_ref[0, 0]
    es = jnp.exp(gs)

    uv = jnp.dot(z.astype(bf), wcat_ref[...].astype(bf),
                 preferred_element_type=jnp.float32)
    uv = jnp.maximum(uv + bcat_ref[...], 0.0)
    u1 = uv[:, :HID]
    u2 = uv[:, HID:]
    g = jnp.sum(u1 * wg2_ref[...], axis=1) + bg2_ref[0, 0]
    eg = jnp.exp(g)

    cdims = (((0,), (0,)), ((), ()))
    ohb = onehot.astype(bf)
    s1_acc[...] += lax.dot_general(ohb, (es[:, None] * ne).astype(bf), cdims,
                                   preferred_element_type=jnp.float32)
    c1_acc[...] += lax.dot_general(onehot, es[:, None], cdims,
                                   preferred_element_type=jnp.float32)
    s2_acc[...] += lax.dot_general(ohb, (eg[:, None] * u2).astype(bf), cdims,
                                   preferred_element_type=jnp.float32)
    c2_acc[...] += lax.dot_general(onehot, eg[:, None], cdims,
                                   preferred_element_type=jnp.float32)

    @pl.when(i == G - 1)
    def _():
        s1_out[...] = s1_acc[...]
        c1_out[...] = c1_acc[...]
        s2_out[...] = s2_acc[...]
        c2_out[...] = c2_acc[...]


def _heads_body(s1, c1, s2, c2, expe,
                wns, bns, wn2, bn2,
                wd1, bd1, wd2, bd2,
                wl1, bl1, wl2, bl2,
                wi1, bi1, wi2, bi2, out_ref):
    def fc(v, w1, b1, w2, b2):
        hh = jnp.dot(jnp.maximum(v, 0.0), w1[...],
                     preferred_element_type=jnp.float32) + b1[...]
        sg = 1.0 / (1.0 + jnp.exp(-hh))
        return jnp.dot(sg, w2[...], preferred_element_type=jnp.float32) + b2[...]

    r1 = 1.0 / (c1[...] + 1e-16)
    xd = jnp.dot(s1[...], wns[...],
                 preferred_element_type=jnp.float32) * r1 + (c1[...] * r1) * bns[...]
    r2 = 1.0 / (c2[...] + 1e-16)
    xi = jnp.dot(s2[...], wn2[...],
                 preferred_element_type=jnp.float32) * r2 + (c2[...] * r2) * bn2[...]
    o1 = fc(xd, wd1, bd1, wd2, bd2)
    o2 = fc(expe[...], wl1, bl1, wl2, bl2)
    o3 = fc(xi, wi1, bi1, wi2, bi2)
    out_ref[...] = jnp.concatenate([o1, o2, o3], axis=1)


def _f32(shape):
    return jax.ShapeDtypeStruct(shape, jnp.float32)


def kernel(exp, x, edge_index, edge_attr, batch, params):
    p = params

    a2 = jnp.stack([p['a_src'], p['a_dst']], axis=1)
    hs, hd = pl.pallas_call(
        _prep_body,
        out_shape=[_f32((N,)), _f32((N,))],
    )(x, p['Wn'], a2)

    el = pl.pallas_call(
        _el_body,
        grid=(pl.cdiv(E, EB),),
        in_specs=[
            pl.BlockSpec((EB, DE), lambda i: (i, 0)),
            pl.BlockSpec((DM, DE), lambda i: (0, 0)),
            pl.BlockSpec((1, DM), lambda i: (0, 0)),
        ],
        out_specs=pl.BlockSpec((EB,), lambda i: (i,)),
        out_shape=_f32((E,)),
    )(edge_attr, p['We'].T, p['a_e'].reshape(1, DM))

    sc_mesh = plsc.VectorSubcoreMesh(core_axis_name="c", subcore_axis_name="s",
                                     num_cores=2, num_subcores=16)
    logit_fn = pl.kernel(
        _logit_sc,
        out_type=[_f32((E,)), _f32((W * N,))],
        mesh=sc_mesh,
        compiler_params=pltpu.CompilerParams(needs_layout_passes=False),
        scratch_types=[
            pltpu.VMEM((N,), jnp.float32),
            pltpu.VMEM((N,), jnp.float32),
            pltpu.VMEM((C,), jnp.int32),
            pltpu.VMEM((C,), jnp.int32),
            pltpu.VMEM((C,), jnp.float32),
            pltpu.VMEM((C,), jnp.float32),
            pltpu.VMEM((N,), jnp.float32),
            pltpu.SemaphoreType.DMA,
            pltpu.SemaphoreType.DMA,
            pltpu.SemaphoreType.DMA,
        ],
    )
    ew, s_flat = logit_fn(edge_index[0], edge_index[1], hs, hd, el)

    scatter_fn = pl.kernel(
        _scatter_sc,
        out_type=[_f32((N, DF)), _f32((N, DF)),
                  _f32((QR, DF)), _f32((QR, DF))],
        mesh=sc_mesh,
        compiler_params=pltpu.CompilerParams(needs_layout_passes=False),
        scratch_types=[
            pltpu.VMEM((C,), jnp.int32),
            pltpu.VMEM((C,), jnp.int32),
            pltpu.VMEM((C,), jnp.int32),
            pltpu.VMEM((C,), jnp.float32),
            pltpu.VMEM((C, DE), jnp.float32),
            pltpu.VMEM((C, DF), jnp.float32),
            pltpu.VMEM((C, DF), jnp.float32),
            pltpu.VMEM_SHARED((N, DF), jnp.float32),
            pltpu.VMEM_SHARED((QR, DF), jnp.float32),
            pltpu.SemaphoreType.DMA,
            pltpu.SemaphoreType.DMA,
            pltpu.SemaphoreType.DMA,
            pltpu.SemaphoreType.DMA,
            pltpu.SemaphoreType.DMA,
        ],
    )
    p0, p1, q0, q1 = scatter_fn(edge_index[0], edge_index[1], ew, x, edge_attr)

    pe0 = q0[:N // 8].reshape(N, DE)
    pe1 = q1[:N // 8].reshape(N, DE)
    s_t = jnp.transpose(s_flat.reshape(W, N))

    expe, xg = pl.pallas_call(
        _gene_body,
        out_shape=[_f32((B, DGE)), _f32((B, DM))],
    )(exp, p['Wg'], p['bg'].reshape(1, -1), p['Wwg'], p['bwg'].reshape(1, -1))

    wcat = jnp.concatenate([p['Wg1'], p['Wn1']], axis=1)
    bcat = jnp.concatenate([p['bg1'], p['bn1']]).reshape(1, -1)
    batch3 = batch.reshape(G, 1, R)

    s1, c1, s2, c2 = pl.pallas_call(
        _main_body,
        grid=(G,),
        in_specs=[
            pl.BlockSpec((R, DF), lambda i: (i, 0)),
            pl.BlockSpec((R, DF), lambda i: (i, 0)),
            pl.BlockSpec((R, DE), lambda i: (i, 0)),
            pl.BlockSpec((R, DE), lambda i: (i, 0)),
            pl.BlockSpec((R, W), lambda i: (i, 0)),
            pl.BlockSpec((1, 1, R), lambda i: (i, 0, 0)),
            pl.BlockSpec((B, DM), lambda i: (0, 0)),
            pl.BlockSpec((DF, DM), lambda i: (0, 0)),
            pl.BlockSpec((DE, DM), lambda i: (0, 0)),
            pl.BlockSpec((1, DM), lambda i: (0, 0)),
            pl.BlockSpec((1, 1), lambda i: (0, 0)),
            pl.BlockSpec((DM, 2 * HID), lambda i: (0, 0)),
            pl.BlockSpec((1, 2 * HID), lambda i: (0, 0)),
            pl.BlockSpec((1, HID), lambda i: (0, 0)),
            pl.BlockSpec((1, 1), lambda i: (0, 0)),
        ],
        out_specs=[
            pl.BlockSpec((B, DM), lambda i: (0, 0)),
            pl.BlockSpec((B, 1), lambda i: (0, 0)),
            pl.BlockSpec((B, HID), lambda i: (0, 0)),
            pl.BlockSpec((B, 1), lambda i: (0, 0)),
        ],
        out_shape=[_f32((B, DM)), _f32((B, 1)), _f32((B, HID)), _f32((B, 1))],
        scratch_shapes=[
            pltpu.VMEM((B, DM), jnp.float32),
            pltpu.VMEM((B, 1), jnp.float32),
            pltpu.VMEM((B, HID), jnp.float32),
            pltpu.VMEM((B, 1), jnp.float32),
        ],
    )(p0, p1, pe0, pe1, s_t,
      batch3, xg, p['Wn'], p['We'],
      p['Wgs'].reshape(1, -1), p['bgs'].reshape(1, 1),
      wcat, bcat, p['Wg2'].reshape(1, -1), p['bg2'].reshape(1, 1))

    out = pl.pallas_call(
        _heads_body,
        out_shape=_f32((B, 3)),
    )(s1, c1, s2, c2, expe,
      p['Wns'], p['bns'].reshape(1, -1), p['Wn2'], p['bn2'].reshape(1, -1),
      p['Wd1'], p['bd1'].reshape(1, -1), p['Wd2'], p['bd2'].reshape(1, 1),
      p['Wl1'], p['bl1'].reshape(1, -1), p['Wl2'], p['bl2'].reshape(1, 1),
      p['Wi1'], p['bi1'].reshape(1, -1), p['Wi2'], p['bi2'].reshape(1, 1))
    return out


# speedup vs baseline: 16.9960x; 1.0893x over previous
import jax
import jax.numpy as jnp
from jax import lax
from jax.experimental import pallas as pl
from jax.experimental.pallas import tpu as pltpu
from jax.experimental.pallas import tpu_sc as plsc

N = 10000
E = 320000
B = 128
DF = 128
DE = 16
DM = 1024
DG = 2089
DGE = 512
HID = 512
OUT = 256

W = 32
EPW = E // W
C = 80
NCH = EPW // C
QR = 1280
EB = 32768

R = 1000
G = N // R


def _prep_body(x_ref, wn_ref, a2_ref, hs_ref, hd_ref):
    w2 = jnp.dot(wn_ref[...], a2_ref[...], preferred_element_type=jnp.float32)
    h2 = lax.dot_general(w2, x_ref[...], (((0,), (1,)), ((), ())),
                         preferred_element_type=jnp.float32)
    hs_ref[...] = h2[0]
    hd_ref[...] = h2[1]


def _el_body(ea_ref, wet_ref, aer_ref, el_ref):
    wea = jnp.dot(aer_ref[...], wet_ref[...], preferred_element_type=jnp.float32)
    el_ref[...] = jnp.sum(ea_ref[...] * wea, axis=1)


def _logit_sc(esrc_hbm, edst_hbm, hs_hbm, hd_hbm, el_hbm, ea_hbm,
              e_out, s_out, q0_out, q1_out,
              hs_v, hd_v, src_v, dst_v, el_v, e_v, s_acc, ea_v, qrow_v, qdst_v,
              q_sp, sem_a, sem_b, sem_c, sem_d):
    cid = lax.axis_index("c")
    sid = lax.axis_index("s")
    wid = sid * 2 + cid
    ebase = wid * EPW

    pltpu.sync_copy(hs_hbm, hs_v)
    pltpu.sync_copy(hd_hbm, hd_v)

    z16 = jnp.zeros((16,), jnp.float32)

    def zs(i, c):
        s_acc[pl.ds(i * 16, 16)] = z16
        return c
    lax.fori_loop(0, N // 16, zs, 0)

    def zq(i, c):
        for f in range(8):
            qrow_v[i, pl.ds(f * 16, 16)] = z16
        return c
    lax.fori_loop(0, C, zq, 0)

    pltpu.sync_copy(qrow_v, q_sp.at[pl.ds(sid * C, C)])
    plsc.subcore_barrier()

    def chunk(k, c):
        off = ebase + k * C
        cp_a = pltpu.async_copy(esrc_hbm.at[pl.ds(off, C)], src_v, sem_a)
        cp_b = pltpu.async_copy(edst_hbm.at[pl.ds(off, C)], dst_v, sem_b)
        cp_c = pltpu.async_copy(el_hbm.at[pl.ds(off, C)], el_v, sem_c)
        cp_d = pltpu.async_copy(ea_hbm.at[pl.ds(off, C)], ea_v, sem_d)
        cp_a.wait()
        cp_b.wait()
        cp_c.wait()
        cp_d.wait()
        for j in range(C // 16):
            sl = pl.ds(j * 16, 16)
            ids = src_v[sl]
            idd = dst_v[sl]
            gs = plsc.load_gather(hs_v, [ids])
            gd = plsc.load_gather(hd_v, [idd])
            logit = gs + gd + el_v[sl]
            logit = jnp.where(logit >= 0.0, logit, 0.2 * logit)
            e = jnp.exp(logit)
            e_v[sl] = e
            plsc.addupdate_scatter(s_acc, [idd], e)
            qdst_v[sl] = lax.shift_right_logical(idd, 3)
            for k2 in range(16):
                i = j * 16 + k2
                base = (idd[k2] & 7) * 16
                qrow_v[i, pl.ds(base, 16)] = (
                    ea_v[i, :] * jnp.full((16,), e[k2], jnp.float32))
        pltpu.sync_copy(e_v, e_out.at[pl.ds(off, C)])
        pltpu.sync_copy(qrow_v, q_sp.at[qdst_v], add=True)

        def unscale(j, c2):
            dv16 = dst_v[pl.ds(j * 16, 16)]
            for k2 in range(16):
                base = (dv16[k2] & 7) * 16
                qrow_v[j * 16 + k2, pl.ds(base, 16)] = z16
            return c2
        lax.fori_loop(0, C // 16, unscale, 0)
        return c
    lax.fori_loop(0, NCH, chunk, 0)

    pltpu.sync_copy(s_acc, s_out.at[pl.ds(wid * N, N)])

    plsc.subcore_barrier()

    pltpu.sync_copy(q_sp.at[pl.ds(sid * C, C)], qrow_v)

    @pl.when(cid == 0)
    def _():
        pltpu.sync_copy(qrow_v, q0_out.at[pl.ds(sid * C, C)])

    @pl.when(cid == 1)
    def _():
        pltpu.sync_copy(qrow_v, q1_out.at[pl.ds(sid * C, C)])


def _scatter_sc(esrc_hbm, edst_hbm, e_hbm, x_hbm,
                p0_out, p1_out,
                src0_v, src1_v, dst_v, e_v, rows0_v, rows1_v,
                p_sp, semr0, semr1, sem_a, sem_b, sem_c):
    cid = lax.axis_index("c")
    sid = lax.axis_index("s")
    ebase = (sid * 2 + cid) * EPW

    z16 = jnp.zeros((16,), jnp.float32)

    def zr(i, c):
        for f in range(8):
            rows0_v[i, pl.ds(f * 16, 16)] = z16
        return c
    lax.fori_loop(0, C, zr, 0)

    def zcp(t, c):
        @pl.when(t % 16 == sid)
        def _():
            pltpu.sync_copy(rows0_v, p_sp.at[pl.ds(t * C, C)])
        return c
    lax.fori_loop(0, N // C, zcp, 0)

    plsc.subcore_barrier()

    def half(k, my_src, my_rows, my_sem, nxt_src, nxt_rows, nxt_sem,
             issue_next):
        cp_b = pltpu.async_copy(edst_hbm.at[pl.ds(ebase + k * C, C)],
                                dst_v, sem_b)
        cp_c = pltpu.async_copy(e_hbm.at[pl.ds(ebase + k * C, C)], e_v, sem_c)
        if issue_next:
            cp_a = pltpu.async_copy(
                esrc_hbm.at[pl.ds(ebase + (k + 1) * C, C)], nxt_src, sem_a)
            cp_a.wait()
            pltpu.async_copy(x_hbm.at[nxt_src], nxt_rows, nxt_sem)
        cp_b.wait()
        cp_c.wait()
        pltpu.make_async_copy(x_hbm.at[my_src], my_rows, my_sem).wait()

        def scale(j, c2):
            ev16 = e_v[pl.ds(j * 16, 16)]
            for k2 in range(16):
                i = j * 16 + k2
                ev = jnp.full((16,), ev16[k2], jnp.float32)
                for f in range(8):
                    my_rows[i, pl.ds(f * 16, 16)] = (
                        my_rows[i, pl.ds(f * 16, 16)] * ev)
            return c2
        lax.fori_loop(0, C // 16, scale, 0)

        pltpu.sync_copy(my_rows, p_sp.at[dst_v], add=True)

    cp0 = pltpu.async_copy(esrc_hbm.at[pl.ds(ebase, C)], src0_v, sem_a)
    cp0.wait()
    pltpu.async_copy(x_hbm.at[src0_v], rows0_v, semr0)

    def pair(kk, c):
        half(kk * 2, src0_v, rows0_v, semr0, src1_v, rows1_v, semr1, True)
        half(kk * 2 + 1, src1_v, rows1_v, semr1, src0_v, rows0_v, semr0, True)
        return c
    lax.fori_loop(0, NCH // 2, pair, 0)
    half(NCH - 1, src0_v, rows0_v, semr0, src1_v, rows1_v, semr1, False)

    plsc.subcore_barrier()

    def drain(t, c):
        @pl.when(t % 16 == sid)
        def _():
            r0 = t * C
            pltpu.sync_copy(p_sp.at[pl.ds(r0, C)], rows0_v)

            @pl.when(cid == 0)
            def _():
                pltpu.sync_copy(rows0_v, p0_out.at[pl.ds(r0, C)])

            @pl.when(cid == 1)
            def _():
                pltpu.sync_copy(rows0_v, p1_out.at[pl.ds(r0, C)])
        return c
    lax.fori_loop(0, N // C, drain, 0)


def _gene_body(exp_ref, wg_ref, bg_ref, wwg_ref, bwg_ref, expe_ref, xg_ref):
    ee = jnp.dot(exp_ref[...], wg_ref[...],
                 preferred_element_type=jnp.float32) + bg_ref[...]
    expe_ref[...] = ee
    xg_ref[...] = jnp.dot(jnp.maximum(ee, 0.0), wwg_ref[...],
                          preferred_element_type=jnp.float32) + bwg_ref[...]


def _main_body(p0_ref, p1_ref, pe0_ref, pe1_ref, st_ref,
               batch_ref, xg_ref, wn_ref, we_ref,
               wgs_ref, bgs_ref, wcat_ref, bcat_ref, wg2_ref, bg2_ref,
               s1_out, c1_out, s2_out, c2_out,
               s1_acc, c1_acc, s2_acc, c2_acc):
    i = pl.program_id(0)

    @pl.when(i == 0)
    def _():
        s1_acc[...] = jnp.zeros_like(s1_acc)
        c1_acc[...] = jnp.zeros_like(c1_acc)
        s2_acc[...] = jnp.zeros_like(s2_acc)
        c2_acc[...] = jnp.zeros_like(c2_acc)

    pmsg = p0_ref[...] + p1_ref[...]
    pemsg = pe0_ref[...] + pe1_ref[...]
    s = jnp.sum(st_ref[...], axis=1)
    inv = 1.0 / (s + 1e-16)
    bf = jnp.bfloat16
    ne = jnp.dot(pmsg.astype(bf), wn_ref[...].astype(bf),
                 preferred_element_type=jnp.float32)
    ne = ne + jnp.dot(pemsg.astype(bf), we_ref[...].astype(bf),
                      preferred_element_type=jnp.float32)
    ne = ne * inv[:, None]
    ne = jnp.where(ne > 0.0, ne, jnp.exp(jnp.minimum(ne, 0.0)) - 1.0)

    b = batch_ref[0, 0, :]
    onehot = (b[:, None] == lax.broadcasted_iota(jnp.int32, (R, B), 1)
              ).astype(jnp.float32)
    z = ne + jnp.dot(onehot.astype(bf), xg_ref[...].astype(bf),
                     preferred_element_type=jnp.float32)

    gs = jnp.sum(ne * wgs_ref[...], axis=1) + bgs_ref[0, 0]
    es = jnp.exp(gs)

    uv = jnp.dot(z.astype(bf), wcat_ref[...].astype(bf),
                 preferred_element_type=jnp.float32)
    uv = jnp.maximum(uv + bcat_ref[...], 0.0)
    u1 = uv[:, :HID]
    u2 = uv[:, HID:]
    g = jnp.sum(u1 * wg2_ref[...], axis=1) + bg2_ref[0, 0]
    eg = jnp.exp(g)

    cdims = (((0,), (0,)), ((), ()))
    ohb = onehot.astype(bf)
    s1_acc[...] += lax.dot_general(ohb, (es[:, None] * ne).astype(bf), cdims,
                                   preferred_element_type=jnp.float32)
    c1_acc[...] += lax.dot_general(onehot, es[:, None], cdims,
                                   preferred_element_type=jnp.float32)
    s2_acc[...] += lax.dot_general(ohb, (eg[:, None] * u2).astype(bf), cdims,
                                   preferred_element_type=jnp.float32)
    c2_acc[...] += lax.dot_general(onehot, eg[:, None], cdims,
                                   preferred_element_type=jnp.float32)

    @pl.when(i == G - 1)
    def _():
        s1_out[...] = s1_acc[...]
        c1_out[...] = c1_acc[...]
        s2_out[...] = s2_acc[...]
        c2_out[...] = c2_acc[...]


def _heads_body(s1, c1, s2, c2, expe,
                wns, bns, wn2, bn2,
                wd1, bd1, wd2, bd2,
                wl1, bl1, wl2, bl2,
                wi1, bi1, wi2, bi2, out_ref):
    def fc(v, w1, b1, w2, b2):
        hh = jnp.dot(jnp.maximum(v, 0.0), w1[...],
                     preferred_element_type=jnp.float32) + b1[...]
        sg = 1.0 / (1.0 + jnp.exp(-hh))
        return jnp.dot(sg, w2[...], preferred_element_type=jnp.float32) + b2[...]

    r1 = 1.0 / (c1[...] + 1e-16)
    xd = jnp.dot(s1[...], wns[...],
                 preferred_element_type=jnp.float32) * r1 + (c1[...] * r1) * bns[...]
    r2 = 1.0 / (c2[...] + 1e-16)
    xi = jnp.dot(s2[...], wn2[...],
                 preferred_element_type=jnp.float32) * r2 + (c2[...] * r2) * bn2[...]
    o1 = fc(xd, wd1, bd1, wd2, bd2)
    o2 = fc(expe[...], wl1, bl1, wl2, bl2)
    o3 = fc(xi, wi1, bi1, wi2, bi2)
    out_ref[...] = jnp.concatenate([o1, o2, o3], axis=1)


def _f32(shape):
    return jax.ShapeDtypeStruct(shape, jnp.float32)


def kernel(exp, x, edge_index, edge_attr, batch, params):
    p = params

    a2 = jnp.stack([p['a_src'], p['a_dst']], axis=1)
    hs, hd = pl.pallas_call(
        _prep_body,
        out_shape=[_f32((N,)), _f32((N,))],
    )(x, p['Wn'], a2)

    el = pl.pallas_call(
        _el_body,
        grid=(pl.cdiv(E, EB),),
        in_specs=[
            pl.BlockSpec((EB, DE), lambda i: (i, 0)),
            pl.BlockSpec((DM, DE), lambda i: (0, 0)),
            pl.BlockSpec((1, DM), lambda i: (0, 0)),
        ],
        out_specs=pl.BlockSpec((EB,), lambda i: (i,)),
        out_shape=_f32((E,)),
    )(edge_attr, p['We'].T, p['a_e'].reshape(1, DM))

    sc_mesh = plsc.VectorSubcoreMesh(core_axis_name="c", subcore_axis_name="s",
                                     num_cores=2, num_subcores=16)
    logit_fn = pl.kernel(
        _logit_sc,
        out_type=[_f32((E,)), _f32((W * N,)),
                  _f32((QR, DF)), _f32((QR, DF))],
        mesh=sc_mesh,
        compiler_params=pltpu.CompilerParams(needs_layout_passes=False),
        scratch_types=[
            pltpu.VMEM((N,), jnp.float32),
            pltpu.VMEM((N,), jnp.float32),
            pltpu.VMEM((C,), jnp.int32),
            pltpu.VMEM((C,), jnp.int32),
            pltpu.VMEM((C,), jnp.float32),
            pltpu.VMEM((C,), jnp.float32),
            pltpu.VMEM((N,), jnp.float32),
            pltpu.VMEM((C, DE), jnp.float32),
            pltpu.VMEM((C, DF), jnp.float32),
            pltpu.VMEM((C,), jnp.int32),
            pltpu.VMEM_SHARED((QR, DF), jnp.float32),
            pltpu.SemaphoreType.DMA,
            pltpu.SemaphoreType.DMA,
            pltpu.SemaphoreType.DMA,
            pltpu.SemaphoreType.DMA,
        ],
    )
    ew, s_flat, q0, q1 = logit_fn(
        edge_index[0], edge_index[1], hs, hd, el, edge_attr)

    scatter_fn = pl.kernel(
        _scatter_sc,
        out_type=[_f32((N, DF)), _f32((N, DF))],
        mesh=sc_mesh,
        compiler_params=pltpu.CompilerParams(needs_layout_passes=False),
        scratch_types=[
            pltpu.VMEM((C,), jnp.int32),
            pltpu.VMEM((C,), jnp.int32),
            pltpu.VMEM((C,), jnp.int32),
            pltpu.VMEM((C,), jnp.float32),
            pltpu.VMEM((C, DF), jnp.float32),
            pltpu.VMEM((C, DF), jnp.float32),
            pltpu.VMEM_SHARED((N, DF), jnp.float32),
            pltpu.SemaphoreType.DMA,
            pltpu.SemaphoreType.DMA,
            pltpu.SemaphoreType.DMA,
            pltpu.SemaphoreType.DMA,
            pltpu.SemaphoreType.DMA,
        ],
    )
    p0, p1 = scatter_fn(edge_index[0], edge_index[1], ew, x)

    pe0 = q0[:N // 8].reshape(N, DE)
    pe1 = q1[:N // 8].reshape(N, DE)
    s_t = jnp.transpose(s_flat.reshape(W, N))

    expe, xg = pl.pallas_call(
        _gene_body,
        out_shape=[_f32((B, DGE)), _f32((B, DM))],
    )(exp, p['Wg'], p['bg'].reshape(1, -1), p['Wwg'], p['bwg'].reshape(1, -1))

    wcat = jnp.concatenate([p['Wg1'], p['Wn1']], axis=1)
    bcat = jnp.concatenate([p['bg1'], p['bn1']]).reshape(1, -1)
    batch3 = batch.reshape(G, 1, R)

    s1, c1, s2, c2 = pl.pallas_call(
        _main_body,
        grid=(G,),
        in_specs=[
            pl.BlockSpec((R, DF), lambda i: (i, 0)),
            pl.BlockSpec((R, DF), lambda i: (i, 0)),
            pl.BlockSpec((R, DE), lambda i: (i, 0)),
            pl.BlockSpec((R, DE), lambda i: (i, 0)),
            pl.BlockSpec((R, W), lambda i: (i, 0)),
            pl.BlockSpec((1, 1, R), lambda i: (i, 0, 0)),
            pl.BlockSpec((B, DM), lambda i: (0, 0)),
            pl.BlockSpec((DF, DM), lambda i: (0, 0)),
            pl.BlockSpec((DE, DM), lambda i: (0, 0)),
            pl.BlockSpec((1, DM), lambda i: (0, 0)),
            pl.BlockSpec((1, 1), lambda i: (0, 0)),
            pl.BlockSpec((DM, 2 * HID), lambda i: (0, 0)),
            pl.BlockSpec((1, 2 * HID), lambda i: (0, 0)),
            pl.BlockSpec((1, HID), lambda i: (0, 0)),
            pl.BlockSpec((1, 1), lambda i: (0, 0)),
        ],
        out_specs=[
            pl.BlockSpec((B, DM), lambda i: (0, 0)),
            pl.BlockSpec((B, 1), lambda i: (0, 0)),
            pl.BlockSpec((B, HID), lambda i: (0, 0)),
            pl.BlockSpec((B, 1), lambda i: (0, 0)),
        ],
        out_shape=[_f32((B, DM)), _f32((B, 1)), _f32((B, HID)), _f32((B, 1))],
        scratch_shapes=[
            pltpu.VMEM((B, DM), jnp.float32),
            pltpu.VMEM((B, 1), jnp.float32),
            pltpu.VMEM((B, HID), jnp.float32),
            pltpu.VMEM((B, 1), jnp.float32),
        ],
    )(p0, p1, pe0, pe1, s_t,
      batch3, xg, p['Wn'], p['We'],
      p['Wgs'].reshape(1, -1), p['bgs'].reshape(1, 1),
      wcat, bcat, p['Wg2'].reshape(1, -1), p['bg2'].reshape(1, 1))

    out = pl.pallas_call(
        _heads_body,
        out_shape=_f32((B, 3)),
    )(s1, c1, s2, c2, expe,
      p['Wns'], p['bns'].reshape(1, -1), p['Wn2'], p['bn2'].reshape(1, -1),
      p['Wd1'], p['bd1'].reshape(1, -1), p['Wd2'], p['bd2'].reshape(1, 1),
      p['Wl1'], p['bl1'].reshape(1, -1), p['Wl2'], p['bl2'].reshape(1, 1),
      p['Wi1'], p['bi1'].reshape(1, -1), p['Wi2'], p['bi2'].reshape(1, 1))
    return out


# speedup vs baseline: 18.5304x; 1.0903x over previous
import jax
import jax.numpy as jnp
from jax import lax
from jax.experimental import pallas as pl
from jax.experimental.pallas import tpu as pltpu
from jax.experimental.pallas import tpu_sc as plsc

N = 10000
E = 320000
B = 128
DF = 128
DE = 16
DM = 1024
DG = 2089
DGE = 512
HID = 512
OUT = 256

W = 32
EPW = E // W
C = 80
NCH = EPW // C
QR = 1280
EB = 32768

R = 1000
G = N // R


def _prep_body(x_ref, wn_ref, a2_ref, hs_ref, hd_ref):
    w2 = jnp.dot(wn_ref[...], a2_ref[...], preferred_element_type=jnp.float32)
    h2 = lax.dot_general(w2, x_ref[...], (((0,), (1,)), ((), ())),
                         preferred_element_type=jnp.float32)
    hs_ref[...] = h2[0]
    hd_ref[...] = h2[1]


def _el_body(ea_ref, wet_ref, aer_ref, el_ref):
    wea = jnp.dot(aer_ref[...], wet_ref[...], preferred_element_type=jnp.float32)
    el_ref[...] = jnp.sum(ea_ref[...] * wea, axis=1)


def _logit_sc(esrc_hbm, edst_hbm, hs_hbm, hd_hbm, el_hbm, ea_hbm,
              e_out, s_out, q0_out, q1_out,
              hs_v, hd_v, src_v, el_v, e_v, s_acc, ea_v,
              dst0_v, dst1_v, qrow0_v, qrow1_v, qdst0_v, qdst1_v,
              q_sp, sem_a, sem_b, sem_c, sem_d, sem_q0, sem_q1):
    cid = lax.axis_index("c")
    sid = lax.axis_index("s")
    wid = sid * 2 + cid
    ebase = wid * EPW

    pltpu.sync_copy(hs_hbm, hs_v)
    pltpu.sync_copy(hd_hbm, hd_v)

    z16 = jnp.zeros((16,), jnp.float32)

    def zs(i, c):
        s_acc[pl.ds(i * 16, 16)] = z16
        return c
    lax.fori_loop(0, N // 16, zs, 0)

    def zq(i, c):
        for f in range(8):
            qrow0_v[i, pl.ds(f * 16, 16)] = z16
            qrow1_v[i, pl.ds(f * 16, 16)] = z16
        return c
    lax.fori_loop(0, C, zq, 0)

    pltpu.sync_copy(qrow0_v, q_sp.at[pl.ds(sid * C, C)])
    plsc.subcore_barrier()

    def unscale(qrow_b, dst_b):
        def go(j, c2):
            dv16 = dst_b[pl.ds(j * 16, 16)]
            for k2 in range(16):
                base = (dv16[k2] & 7) * 16
                qrow_b[j * 16 + k2, pl.ds(base, 16)] = z16
            return c2
        lax.fori_loop(0, C // 16, go, 0)

    def half(k, dst_b, qrow_b, qdst_b, sem_qb, wait_prev):
        off = ebase + k * C
        cp_a = pltpu.async_copy(esrc_hbm.at[pl.ds(off, C)], src_v, sem_a)
        cp_c = pltpu.async_copy(el_hbm.at[pl.ds(off, C)], el_v, sem_c)
        cp_d = pltpu.async_copy(ea_hbm.at[pl.ds(off, C)], ea_v, sem_d)
        if wait_prev:
            pltpu.make_async_copy(qrow_b, q_sp.at[qdst_b], sem_qb).wait()
            unscale(qrow_b, dst_b)
        cp_b = pltpu.async_copy(edst_hbm.at[pl.ds(off, C)], dst_b, sem_b)
        cp_a.wait()
        cp_b.wait()
        cp_c.wait()
        cp_d.wait()
        for j in range(C // 16):
            sl = pl.ds(j * 16, 16)
            ids = src_v[sl]
            idd = dst_b[sl]
            gs = plsc.load_gather(hs_v, [ids])
            gd = plsc.load_gather(hd_v, [idd])
            logit = gs + gd + el_v[sl]
            logit = jnp.where(logit >= 0.0, logit, 0.2 * logit)
            e = jnp.exp(logit)
            e_v[sl] = e
            plsc.addupdate_scatter(s_acc, [idd], e)
            qdst_b[sl] = lax.shift_right_logical(idd, 3)
            for k2 in range(16):
                i = j * 16 + k2
                base = (idd[k2] & 7) * 16
                qrow_b[i, pl.ds(base, 16)] = (
                    ea_v[i, :] * jnp.full((16,), e[k2], jnp.float32))
        pltpu.sync_copy(e_v, e_out.at[pl.ds(off, C)])
        pltpu.async_copy(qrow_b, q_sp.at[qdst_b], sem_qb, add=True)

    half(0, dst0_v, qrow0_v, qdst0_v, sem_q0, False)
    half(1, dst1_v, qrow1_v, qdst1_v, sem_q1, False)

    def pair(kk, c):
        half(kk * 2, dst0_v, qrow0_v, qdst0_v, sem_q0, True)
        half(kk * 2 + 1, dst1_v, qrow1_v, qdst1_v, sem_q1, True)
        return c
    lax.fori_loop(1, NCH // 2, pair, 0)
    half(NCH - 1, dst0_v, qrow0_v, qdst0_v, sem_q0, True)

    pltpu.make_async_copy(qrow1_v, q_sp.at[qdst1_v], sem_q1).wait()
    pltpu.make_async_copy(qrow0_v, q_sp.at[qdst0_v], sem_q0).wait()

    pltpu.sync_copy(s_acc, s_out.at[pl.ds(wid * N, N)])

    plsc.subcore_barrier()

    pltpu.sync_copy(q_sp.at[pl.ds(sid * C, C)], qrow0_v)

    @pl.when(cid == 0)
    def _():
        pltpu.sync_copy(qrow0_v, q0_out.at[pl.ds(sid * C, C)])

    @pl.when(cid == 1)
    def _():
        pltpu.sync_copy(qrow0_v, q1_out.at[pl.ds(sid * C, C)])


def _scatter_sc(esrc_hbm, edst_hbm, e_hbm, x_hbm,
                p0_out, p1_out,
                src0_v, src1_v, dst_v, e_v, rows0_v, rows1_v,
                p_sp, semr0, semr1, sem_a, sem_b, sem_c):
    cid = lax.axis_index("c")
    sid = lax.axis_index("s")
    ebase = (sid * 2 + cid) * EPW

    z16 = jnp.zeros((16,), jnp.float32)

    def zr(i, c):
        for f in range(8):
            rows0_v[i, pl.ds(f * 16, 16)] = z16
        return c
    lax.fori_loop(0, C, zr, 0)

    def zcp(t, c):
        @pl.when(t % 16 == sid)
        def _():
            pltpu.sync_copy(rows0_v, p_sp.at[pl.ds(t * C, C)])
        return c
    lax.fori_loop(0, N // C, zcp, 0)

    plsc.subcore_barrier()

    def half(k, my_src, my_rows, my_sem, nxt_src, nxt_rows, nxt_sem,
             issue_next):
        cp_b = pltpu.async_copy(edst_hbm.at[pl.ds(ebase + k * C, C)],
                                dst_v, sem_b)
        cp_c = pltpu.async_copy(e_hbm.at[pl.ds(ebase + k * C, C)], e_v, sem_c)
        if issue_next:
            cp_a = pltpu.async_copy(
                esrc_hbm.at[pl.ds(ebase + (k + 1) * C, C)], nxt_src, sem_a)
            cp_a.wait()
            pltpu.async_copy(x_hbm.at[nxt_src], nxt_rows, nxt_sem)
        cp_b.wait()
        cp_c.wait()
        pltpu.make_async_copy(x_hbm.at[my_src], my_rows, my_sem).wait()

        def scale(j, c2):
            ev16 = e_v[pl.ds(j * 16, 16)]
            for k2 in range(16):
                i = j * 16 + k2
                ev = jnp.full((16,), ev16[k2], jnp.float32)
                for f in range(8):
                    my_rows[i, pl.ds(f * 16, 16)] = (
                        my_rows[i, pl.ds(f * 16, 16)] * ev)
            return c2
        lax.fori_loop(0, C // 16, scale, 0)

        pltpu.sync_copy(my_rows, p_sp.at[dst_v], add=True)

    cp0 = pltpu.async_copy(esrc_hbm.at[pl.ds(ebase, C)], src0_v, sem_a)
    cp0.wait()
    pltpu.async_copy(x_hbm.at[src0_v], rows0_v, semr0)

    def pair(kk, c):
        half(kk * 2, src0_v, rows0_v, semr0, src1_v, rows1_v, semr1, True)
        half(kk * 2 + 1, src1_v, rows1_v, semr1, src0_v, rows0_v, semr0, True)
        return c
    lax.fori_loop(0, NCH // 2, pair, 0)
    half(NCH - 1, src0_v, rows0_v, semr0, src1_v, rows1_v, semr1, False)

    plsc.subcore_barrier()

    def drain(t, c):
        @pl.when(t % 16 == sid)
        def _():
            r0 = t * C
            pltpu.sync_copy(p_sp.at[pl.ds(r0, C)], rows0_v)

            @pl.when(cid == 0)
            def _():
                pltpu.sync_copy(rows0_v, p0_out.at[pl.ds(r0, C)])

            @pl.when(cid == 1)
            def _():
                pltpu.sync_copy(rows0_v, p1_out.at[pl.ds(r0, C)])
        return c
    lax.fori_loop(0, N // C, drain, 0)


def _gene_body(exp_ref, wg_ref, bg_ref, wwg_ref, bwg_ref, expe_ref, xg_ref):
    ee = jnp.dot(exp_ref[...], wg_ref[...],
                 preferred_element_type=jnp.float32) + bg_ref[...]
    expe_ref[...] = ee
    xg_ref[...] = jnp.dot(jnp.maximum(ee, 0.0), wwg_ref[...],
                          preferred_element_type=jnp.float32) + bwg_ref[...]


def _main_body(p0_ref, p1_ref, pe0_ref, pe1_ref, st_ref,
               batch_ref, xg_ref, wn_ref, we_ref,
               wgs_ref, bgs_ref, wcat_ref, bcat_ref, wg2_ref, bg2_ref,
               s1_out, c1_out, s2_out, c2_out,
               s1_acc, c1_acc, s2_acc, c2_acc):
    i = pl.program_id(0)

    @pl.when(i == 0)
    def _():
        s1_acc[...] = jnp.zeros_like(s1_acc)
        c1_acc[...] = jnp.zeros_like(c1_acc)
        s2_acc[...] = jnp.zeros_like(s2_acc)
        c2_acc[...] = jnp.zeros_like(c2_acc)

    pmsg = p0_ref[...] + p1_ref[...]
    pemsg = pe0_ref[...] + pe1_ref[...]
    s = jnp.sum(st_ref[...], axis=1)
    inv = 1.0 / (s + 1e-16)
    bf = jnp.bfloat16
    ne = jnp.dot(pmsg.astype(bf), wn_ref[...].astype(bf),
                 preferred_element_type=jnp.float32)
    ne = ne + jnp.dot(pemsg.astype(bf), we_ref[...].astype(bf),
                      preferred_element_type=jnp.float32)
    ne = ne * inv[:, None]
    ne = jnp.where(ne > 0.0, ne, jnp.exp(jnp.minimum(ne, 0.0)) - 1.0)

    b = batch_ref[0, 0, :]
    onehot = (b[:, None] == lax.broadcasted_iota(jnp.int32, (R, B), 1)
              ).astype(jnp.float32)
    z = ne + jnp.dot(onehot.astype(bf), xg_ref[...].astype(bf),
                     preferred_element_type=jnp.float32)

    gs = jnp.sum(ne * wgs_ref[...], axis=1) + bgs_ref[0, 0]
    es = jnp.exp(gs)

    uv = jnp.dot(z.astype(bf), wcat_ref[...].astype(bf),
                 preferred_element_type=jnp.float32)
    uv = jnp.maximum(uv + bcat_ref[...], 0.0)
    u1 = uv[:, :HID]
    u2 = uv[:, HID:]
    g = jnp.sum(u1 * wg2_ref[...], axis=1) + bg2_ref[0, 0]
    eg = jnp.exp(g)

    cdims = (((0,), (0,)), ((), ()))
    ohb = onehot.astype(bf)
    s1_acc[...] += lax.dot_general(ohb, (es[:, None] * ne).astype(bf), cdims,
                                   preferred_element_type=jnp.float32)
    c1_acc[...] += lax.dot_general(onehot, es[:, None], cdims,
                                   preferred_element_type=jnp.float32)
    s2_acc[...] += lax.dot_general(ohb, (eg[:, None] * u2).astype(bf), cdims,
                                   preferred_element_type=jnp.float32)
    c2_acc[...] += lax.dot_general(onehot, eg[:, None], cdims,
                                   preferred_element_type=jnp.float32)

    @pl.when(i == G - 1)
    def _():
        s1_out[...] = s1_acc[...]
        c1_out[...] = c1_acc[...]
        s2_out[...] = s2_acc[...]
        c2_out[...] = c2_acc[...]


def _heads_body(s1, c1, s2, c2, expe,
                wns, bns, wn2, bn2,
                wd1, bd1, wd2, bd2,
                wl1, bl1, wl2, bl2,
                wi1, bi1, wi2, bi2, out_ref):
    def fc(v, w1, b1, w2, b2):
        hh = jnp.dot(jnp.maximum(v, 0.0), w1[...],
                     preferred_element_type=jnp.float32) + b1[...]
        sg = 1.0 / (1.0 + jnp.exp(-hh))
        return jnp.dot(sg, w2[...], preferred_element_type=jnp.float32) + b2[...]

    r1 = 1.0 / (c1[...] + 1e-16)
    xd = jnp.dot(s1[...], wns[...],
                 preferred_element_type=jnp.float32) * r1 + (c1[...] * r1) * bns[...]
    r2 = 1.0 / (c2[...] + 1e-16)
    xi = jnp.dot(s2[...], wn2[...],
                 preferred_element_type=jnp.float32) * r2 + (c2[...] * r2) * bn2[...]
    o1 = fc(xd, wd1, bd1, wd2, bd2)
    o2 = fc(expe[...], wl1, bl1, wl2, bl2)
    o3 = fc(xi, wi1, bi1, wi2, bi2)
    out_ref[...] = jnp.concatenate([o1, o2, o3], axis=1)


def _f32(shape):
    return jax.ShapeDtypeStruct(shape, jnp.float32)


def kernel(exp, x, edge_index, edge_attr, batch, params):
    p = params

    a2 = jnp.stack([p['a_src'], p['a_dst']], axis=1)
    hs, hd = pl.pallas_call(
        _prep_body,
        out_shape=[_f32((N,)), _f32((N,))],
    )(x, p['Wn'], a2)

    el = pl.pallas_call(
        _el_body,
        grid=(pl.cdiv(E, EB),),
        in_specs=[
            pl.BlockSpec((EB, DE), lambda i: (i, 0)),
            pl.BlockSpec((DM, DE), lambda i: (0, 0)),
            pl.BlockSpec((1, DM), lambda i: (0, 0)),
        ],
        out_specs=pl.BlockSpec((EB,), lambda i: (i,)),
        out_shape=_f32((E,)),
    )(edge_attr, p['We'].T, p['a_e'].reshape(1, DM))

    sc_mesh = plsc.VectorSubcoreMesh(core_axis_name="c", subcore_axis_name="s",
                                     num_cores=2, num_subcores=16)
    logit_fn = pl.kernel(
        _logit_sc,
        out_type=[_f32((E,)), _f32((W * N,)),
                  _f32((QR, DF)), _f32((QR, DF))],
        mesh=sc_mesh,
        compiler_params=pltpu.CompilerParams(needs_layout_passes=False),
        scratch_types=[
            pltpu.VMEM((N,), jnp.float32),
            pltpu.VMEM((N,), jnp.float32),
            pltpu.VMEM((C,), jnp.int32),
            pltpu.VMEM((C,), jnp.float32),
            pltpu.VMEM((C,), jnp.float32),
            pltpu.VMEM((N,), jnp.float32),
            pltpu.VMEM((C, DE), jnp.float32),
            pltpu.VMEM((C,), jnp.int32),
            pltpu.VMEM((C,), jnp.int32),
            pltpu.VMEM((C, DF), jnp.float32),
            pltpu.VMEM((C, DF), jnp.float32),
            pltpu.VMEM((C,), jnp.int32),
            pltpu.VMEM((C,), jnp.int32),
            pltpu.VMEM_SHARED((QR, DF), jnp.float32),
            pltpu.SemaphoreType.DMA,
            pltpu.SemaphoreType.DMA,
            pltpu.SemaphoreType.DMA,
            pltpu.SemaphoreType.DMA,
            pltpu.SemaphoreType.DMA,
            pltpu.SemaphoreType.DMA,
        ],
    )
    ew, s_flat, q0, q1 = logit_fn(
        edge_index[0], edge_index[1], hs, hd, el, edge_attr)

    scatter_fn = pl.kernel(
        _scatter_sc,
        out_type=[_f32((N, DF)), _f32((N, DF))],
        mesh=sc_mesh,
        compiler_params=pltpu.CompilerParams(needs_layout_passes=False),
        scratch_types=[
            pltpu.VMEM((C,), jnp.int32),
            pltpu.VMEM((C,), jnp.int32),
            pltpu.VMEM((C,), jnp.int32),
            pltpu.VMEM((C,), jnp.float32),
            pltpu.VMEM((C, DF), jnp.float32),
            pltpu.VMEM((C, DF), jnp.float32),
            pltpu.VMEM_SHARED((N, DF), jnp.float32),
            pltpu.SemaphoreType.DMA,
            pltpu.SemaphoreType.DMA,
            pltpu.SemaphoreType.DMA,
            pltpu.SemaphoreType.DMA,
            pltpu.SemaphoreType.DMA,
        ],
    )
    p0, p1 = scatter_fn(edge_index[0], edge_index[1], ew, x)

    pe0 = q0[:N // 8].reshape(N, DE)
    pe1 = q1[:N // 8].reshape(N, DE)
    s_t = jnp.transpose(s_flat.reshape(W, N))

    expe, xg = pl.pallas_call(
        _gene_body,
        out_shape=[_f32((B, DGE)), _f32((B, DM))],
    )(exp, p['Wg'], p['bg'].reshape(1, -1), p['Wwg'], p['bwg'].reshape(1, -1))

    wcat = jnp.concatenate([p['Wg1'], p['Wn1']], axis=1)
    bcat = jnp.concatenate([p['bg1'], p['bn1']]).reshape(1, -1)
    batch3 = batch.reshape(G, 1, R)

    s1, c1, s2, c2 = pl.pallas_call(
        _main_body,
        grid=(G,),
        in_specs=[
            pl.BlockSpec((R, DF), lambda i: (i, 0)),
            pl.BlockSpec((R, DF), lambda i: (i, 0)),
            pl.BlockSpec((R, DE), lambda i: (i, 0)),
            pl.BlockSpec((R, DE), lambda i: (i, 0)),
            pl.BlockSpec((R, W), lambda i: (i, 0)),
            pl.BlockSpec((1, 1, R), lambda i: (i, 0, 0)),
            pl.BlockSpec((B, DM), lambda i: (0, 0)),
            pl.BlockSpec((DF, DM), lambda i: (0, 0)),
            pl.BlockSpec((DE, DM), lambda i: (0, 0)),
            pl.BlockSpec((1, DM), lambda i: (0, 0)),
            pl.BlockSpec((1, 1), lambda i: (0, 0)),
            pl.BlockSpec((DM, 2 * HID), lambda i: (0, 0)),
            pl.BlockSpec((1, 2 * HID), lambda i: (0, 0)),
            pl.BlockSpec((1, HID), lambda i: (0, 0)),
            pl.BlockSpec((1, 1), lambda i: (0, 0)),
        ],
        out_specs=[
            pl.BlockSpec((B, DM), lambda i: (0, 0)),
            pl.BlockSpec((B, 1), lambda i: (0, 0)),
            pl.BlockSpec((B, HID), lambda i: (0, 0)),
            pl.BlockSpec((B, 1), lambda i: (0, 0)),
        ],
        out_shape=[_f32((B, DM)), _f32((B, 1)), _f32((B, HID)), _f32((B, 1))],
        scratch_shapes=[
            pltpu.VMEM((B, DM), jnp.float32),
            pltpu.VMEM((B, 1), jnp.float32),
            pltpu.VMEM((B, HID), jnp.float32),
            pltpu.VMEM((B, 1), jnp.float32),
        ],
    )(p0, p1, pe0, pe1, s_t,
      batch3, xg, p['Wn'], p['We'],
      p['Wgs'].reshape(1, -1), p['bgs'].reshape(1, 1),
      wcat, bcat, p['Wg2'].reshape(1, -1), p['bg2'].reshape(1, 1))

    out = pl.pallas_call(
        _heads_body,
        out_shape=_f32((B, 3)),
    )(s1, c1, s2, c2, expe,
      p['Wns'], p['bns'].reshape(1, -1), p['Wn2'], p['bn2'].reshape(1, -1),
      p['Wd1'], p['bd1'].reshape(1, -1), p['Wd2'], p['bd2'].reshape(1, 1),
      p['Wl1'], p['bl1'].reshape(1, -1), p['Wl2'], p['bl2'].reshape(1, 1),
      p['Wi1'], p['bi1'].reshape(1, -1), p['Wi2'], p['bi2'].reshape(1, 1))
    return out
